# Initial kernel scaffold; baseline (speedup 1.0000x reference)
import jax
import jax.numpy as jnp
from jax.experimental import pallas as pl


def kernel(obs_tokens, act_tokens, W_obs, W_act, W_special, PE_obs, PE_act, PE_special, PE_time):
    raise NotImplementedError("write your pallas kernel here")


# speedup vs baseline: 1.7247x; 1.7247x over previous
import functools

import jax
import jax.numpy as jnp
from jax import lax
from jax.experimental import pallas as pl
from jax.experimental.pallas import tpu as pltpu
from jax.experimental.pallas import tpu_sc as plsc

_B, _T, _LO, _LA = 256, 50, 16, 4
_LT = _LO + 1 + _LA
_D = 64
_NW = 32
_TS = _B * _T
_TS_PER_W = _TS // _NW
_G = 8
_NCHUNK = _TS_PER_W // _G
_OBS_C = _G * _LO
_ACT_C = _G * _LA
_OUT_C = _G * _LT

_mesh = plsc.VectorSubcoreMesh(core_axis_name="c", subcore_axis_name="s")


@functools.partial(
    pl.kernel,
    mesh=_mesh,
    compiler_params=pltpu.CompilerParams(use_tc_tiling_on_sc=False),
    out_type=jax.ShapeDtypeStruct((_TS * _LT, _D), jnp.float32),
    scratch_types=[
        pltpu.VMEM((_OBS_C,), jnp.int32),
        pltpu.VMEM((_ACT_C,), jnp.int32),
        pltpu.VMEM((_OBS_C, _D), jnp.float32),
        pltpu.VMEM((_ACT_C, _D), jnp.float32),
        pltpu.VMEM((_OUT_C, _D), jnp.float32),
        pltpu.VMEM((_LO, _D), jnp.float32),
        pltpu.VMEM((_LA, _D), jnp.float32),
        pltpu.VMEM((_T, _D), jnp.float32),
        pltpu.VMEM((1, _D), jnp.float32),
        pltpu.VMEM((1, _D), jnp.float32),
        pltpu.SemaphoreType.DMA,
        pltpu.SemaphoreType.DMA,
    ],
)
def _episode_sc(obs_idx_hbm, act_idx_hbm, w_obs_hbm, w_act_hbm, w_sp_hbm,
                pe_obs_hbm, pe_act_hbm, pe_sp_hbm, pe_time_hbm, out_hbm,
                idx_obs, idx_act, obs_rows, act_rows, out_buf,
                pe_obs_v, pe_act_v, pe_time_v, sp_v, pesp_v, sem0, sem1):
    wid = lax.axis_index("s") * 2 + lax.axis_index("c")

    pltpu.sync_copy(pe_obs_hbm.at[pl.ds(0, _LO)], pe_obs_v)
    pltpu.sync_copy(pe_act_hbm, pe_act_v)
    pltpu.sync_copy(pe_time_hbm, pe_time_v)
    pltpu.sync_copy(w_sp_hbm.at[pl.ds(0, 1)], sp_v)
    pltpu.sync_copy(pe_sp_hbm, pesp_v)
    for c in range(_D // 16):
        sp_v[0, pl.ds(c * 16, 16)] = (
            sp_v[0, pl.ds(c * 16, 16)] + pesp_v[0, pl.ds(c * 16, 16)]
        )

    obs_base = wid * (_TS_PER_W * _LO)
    act_base = wid * (_TS_PER_W * _LA)
    out_base = wid * (_TS_PER_W * _LT)

    def chunk_body(g, carry):
        pltpu.sync_copy(obs_idx_hbm.at[pl.ds(obs_base + g * _OBS_C, _OBS_C)],
                        idx_obs)
        pltpu.sync_copy(act_idx_hbm.at[pl.ds(act_base + g * _ACT_C, _ACT_C)],
                        idx_act)
        co = pltpu.async_copy(w_obs_hbm.at[idx_obs], obs_rows, sem0)
        ca = pltpu.async_copy(w_act_hbm.at[idx_act], act_rows, sem1)
        co.wait()
        ca.wait()

        def ts_body(j, carry2):
            t = lax.rem(g * _G + j, _T)
            tv = [pe_time_v[t, pl.ds(c * 16, 16)] for c in range(_D // 16)]
            for s in range(_LO):
                for c in range(_D // 16):
                    out_buf[j * _LT + s, pl.ds(c * 16, 16)] = (
                        obs_rows[j * _LO + s, pl.ds(c * 16, 16)]
                        + pe_obs_v[s, pl.ds(c * 16, 16)] + tv[c]
                    )
            for c in range(_D // 16):
                out_buf[j * _LT + _LO, pl.ds(c * 16, 16)] = (
                    sp_v[0, pl.ds(c * 16, 16)] + tv[c]
                )
            for a in range(_LA):
                for c in range(_D // 16):
                    out_buf[j * _LT + _LO + 1 + a, pl.ds(c * 16, 16)] = (
                        act_rows[j * _LA + a, pl.ds(c * 16, 16)]
                        + pe_act_v[a, pl.ds(c * 16, 16)] + tv[c]
                    )
            return carry2

        lax.fori_loop(0, _G, ts_body, 0)
        pltpu.sync_copy(out_buf,
                        out_hbm.at[pl.ds(out_base + g * _OUT_C, _OUT_C)])
        return carry

    lax.fori_loop(0, _NCHUNK, chunk_body, 0)


def kernel(obs_tokens, act_tokens, W_obs, W_act, W_special,
           PE_obs, PE_act, PE_special, PE_time):
    obs_idx = obs_tokens.reshape(-1)
    act_idx = act_tokens.reshape(-1)
    out = _episode_sc(obs_idx, act_idx, W_obs, W_act, W_special,
                      PE_obs, PE_act, PE_special, PE_time)
    return out.reshape(_B, _T * _LT, _D)


# speedup vs baseline: 2.1035x; 1.2196x over previous
import functools

import jax
import jax.numpy as jnp
from jax import lax
from jax.experimental import pallas as pl
from jax.experimental.pallas import tpu as pltpu
from jax.experimental.pallas import tpu_sc as plsc

_B, _T, _LO, _LA = 256, 50, 16, 4
_LT = _LO + 1 + _LA
_D = 64
_NW = 32
_TS = _B * _T
_TS_PER_W = _TS // _NW
_G = 8
_NCHUNK = _TS_PER_W // _G
_OBS_C = _G * _LO
_ACT_C = _G * _LA
_OUT_C = _G * _LT

_mesh = plsc.VectorSubcoreMesh(core_axis_name="c", subcore_axis_name="s")


@functools.partial(
    pl.kernel,
    mesh=_mesh,
    compiler_params=pltpu.CompilerParams(use_tc_tiling_on_sc=False),
    out_type=jax.ShapeDtypeStruct((_TS * _LT, _D), jnp.float32),
    scratch_types=[
        pltpu.VMEM((_NCHUNK, _OBS_C), jnp.int32),
        pltpu.VMEM((_NCHUNK, _ACT_C), jnp.int32),
        pltpu.VMEM((2, _OBS_C, _D), jnp.float32),
        pltpu.VMEM((2, _ACT_C, _D), jnp.float32),
        pltpu.VMEM((2, _OUT_C, _D), jnp.float32),
        pltpu.VMEM((_LO, _D), jnp.float32),
        pltpu.VMEM((_LA, _D), jnp.float32),
        pltpu.VMEM((_T, _D), jnp.float32),
        pltpu.VMEM((1, _D), jnp.float32),
        pltpu.VMEM((1, _D), jnp.float32),
        pltpu.SemaphoreType.DMA,
        pltpu.SemaphoreType.DMA,
        pltpu.SemaphoreType.DMA,
        pltpu.SemaphoreType.DMA,
        pltpu.SemaphoreType.DMA,
        pltpu.SemaphoreType.DMA,
    ],
)
def _episode_sc(obs_idx_hbm, act_idx_hbm, w_obs_hbm, w_act_hbm, w_sp_hbm,
                pe_obs_hbm, pe_act_hbm, pe_sp_hbm, pe_time_hbm, out_hbm,
                idx_obs_v, idx_act_v, obs_rows, act_rows, out_buf,
                pe_obs_v, pe_act_v, pe_time_v, sp_v, pesp_v,
                sem_go0, sem_go1, sem_ga0, sem_ga1, sem_out0, sem_out1):
    wid = lax.axis_index("s") * 2 + lax.axis_index("c")
    sem_go = [sem_go0, sem_go1]
    sem_ga = [sem_ga0, sem_ga1]
    sem_out = [sem_out0, sem_out1]
    out_base = wid * (_TS_PER_W * _LT)

    pltpu.sync_copy(obs_idx_hbm.at[wid], idx_obs_v)
    pltpu.sync_copy(act_idx_hbm.at[wid], idx_act_v)
    pltpu.sync_copy(pe_obs_hbm.at[pl.ds(0, _LO)], pe_obs_v)
    pltpu.sync_copy(pe_act_hbm, pe_act_v)
    pltpu.sync_copy(pe_time_hbm, pe_time_v)
    pltpu.sync_copy(w_sp_hbm.at[pl.ds(0, 1)], sp_v)
    pltpu.sync_copy(pe_sp_hbm, pesp_v)
    for c in range(_D // 16):
        sp_v[0, pl.ds(c * 16, 16)] = (
            sp_v[0, pl.ds(c * 16, 16)] + pesp_v[0, pl.ds(c * 16, 16)]
        )

    def fire(g, slot):
        pltpu.make_async_copy(w_obs_hbm.at[idx_obs_v.at[g]],
                              obs_rows.at[slot], sem_go[slot]).start()
        pltpu.make_async_copy(w_act_hbm.at[idx_act_v.at[g]],
                              act_rows.at[slot], sem_ga[slot]).start()

    def compute_store(g, slot):
        pltpu.make_async_copy(w_obs_hbm.at[idx_obs_v.at[g]],
                              obs_rows.at[slot], sem_go[slot]).wait()
        pltpu.make_async_copy(w_act_hbm.at[idx_act_v.at[g]],
                              act_rows.at[slot], sem_ga[slot]).wait()

        @pl.when(g >= 2)
        def _():
            pltpu.make_async_copy(
                out_buf.at[slot],
                out_hbm.at[pl.ds(out_base, _OUT_C)], sem_out[slot]).wait()

        def ts_body(j, carry):
            t = lax.rem(g * _G + j, _T)
            tv = [pe_time_v[t, pl.ds(c * 16, 16)] for c in range(_D // 16)]
            for s in range(_LO):
                for c in range(_D // 16):
                    out_buf[slot, j * _LT + s, pl.ds(c * 16, 16)] = (
                        obs_rows[slot, j * _LO + s, pl.ds(c * 16, 16)]
                        + pe_obs_v[s, pl.ds(c * 16, 16)] + tv[c]
                    )
            for c in range(_D // 16):
                out_buf[slot, j * _LT + _LO, pl.ds(c * 16, 16)] = (
                    sp_v[0, pl.ds(c * 16, 16)] + tv[c]
                )
            for a in range(_LA):
                for c in range(_D // 16):
                    out_buf[slot, j * _LT + _LO + 1 + a, pl.ds(c * 16, 16)] = (
                        act_rows[slot, j * _LA + a, pl.ds(c * 16, 16)]
                        + pe_act_v[a, pl.ds(c * 16, 16)] + tv[c]
                    )
            return carry

        lax.fori_loop(0, _G, ts_body, 0)
        pltpu.make_async_copy(
            out_buf.at[slot],
            out_hbm.at[pl.ds(out_base + g * _OUT_C, _OUT_C)],
            sem_out[slot]).start()

    fire(0, 0)

    def outer(i, carry):
        g0 = 2 * i
        fire(g0 + 1, 1)
        compute_store(g0, 0)

        @pl.when(g0 + 2 < _NCHUNK)
        def _():
            fire(g0 + 2, 0)

        compute_store(g0 + 1, 1)
        return carry

    lax.fori_loop(0, _NCHUNK // 2, outer, 0)

    for slot in range(2):
        pltpu.make_async_copy(
            out_buf.at[slot],
            out_hbm.at[pl.ds(out_base, _OUT_C)], sem_out[slot]).wait()


def kernel(obs_tokens, act_tokens, W_obs, W_act, W_special,
           PE_obs, PE_act, PE_special, PE_time):
    obs_idx = obs_tokens.reshape(_NW, _NCHUNK, _OBS_C)
    act_idx = act_tokens.reshape(_NW, _NCHUNK, _ACT_C)
    out = _episode_sc(obs_idx, act_idx, W_obs, W_act, W_special,
                      PE_obs, PE_act, PE_special, PE_time)
    return out.reshape(_B, _T * _LT, _D)


# speedup vs baseline: 2.5203x; 1.1982x over previous
import functools

import jax
import jax.numpy as jnp
from jax import lax
from jax.experimental import pallas as pl
from jax.experimental.pallas import tpu as pltpu
from jax.experimental.pallas import tpu_sc as plsc

_B, _T, _LO, _LA = 256, 50, 16, 4
_LT = _LO + 1 + _LA
_D = 64
_NW = 32
_TS = _B * _T
_TS_PER_W = _TS // _NW
_G = 8
_NCHUNK = _TS_PER_W // _G
_OBS_C = _G * _LO
_ACT_C = _G * _LA
_OUT_C = _G * _LT

_mesh = plsc.VectorSubcoreMesh(core_axis_name="c", subcore_axis_name="s")


@functools.partial(
    pl.kernel,
    mesh=_mesh,
    compiler_params=pltpu.CompilerParams(use_tc_tiling_on_sc=False),
    out_type=jax.ShapeDtypeStruct((_TS * _LT, _D), jnp.float32),
    scratch_types=[
        pltpu.VMEM((_TS_PER_W * _LO,), jnp.int32),
        pltpu.VMEM((_TS_PER_W * _LA,), jnp.int32),
        pltpu.VMEM((2, _OBS_C, _D), jnp.float32),
        pltpu.VMEM((2, _ACT_C, _D), jnp.float32),
        pltpu.VMEM((2, _OUT_C, _D), jnp.float32),
        pltpu.VMEM((_LO, _D), jnp.float32),
        pltpu.VMEM((_LA, _D), jnp.float32),
        pltpu.VMEM((_T, _D), jnp.float32),
        pltpu.VMEM((1, _D), jnp.float32),
        pltpu.VMEM((1, _D), jnp.float32),
        pltpu.SemaphoreType.DMA,
        pltpu.SemaphoreType.DMA,
        pltpu.SemaphoreType.DMA,
        pltpu.SemaphoreType.DMA,
        pltpu.SemaphoreType.DMA,
        pltpu.SemaphoreType.DMA,
    ],
)
def _episode_sc(obs_idx_hbm, act_idx_hbm, w_obs_hbm, w_act_hbm, w_sp_hbm,
                pe_obs_hbm, pe_act_hbm, pe_sp_hbm, pe_time_hbm, out_hbm,
                idx_obs_v, idx_act_v, obs_rows, act_rows, out_buf,
                pe_obs_v, pe_act_v, pe_time_v, sp_v, pesp_v,
                sem_go0, sem_go1, sem_ga0, sem_ga1, sem_out0, sem_out1):
    wid = lax.axis_index("s") * 2 + lax.axis_index("c")
    sem_go = [sem_go0, sem_go1]
    sem_ga = [sem_ga0, sem_ga1]
    sem_out = [sem_out0, sem_out1]
    out_base = wid * (_TS_PER_W * _LT)

    pltpu.sync_copy(obs_idx_hbm.at[pl.ds(wid * (_TS_PER_W * _LO),
                                         _TS_PER_W * _LO)], idx_obs_v)
    pltpu.sync_copy(act_idx_hbm.at[pl.ds(wid * (_TS_PER_W * _LA),
                                         _TS_PER_W * _LA)], idx_act_v)
    pltpu.sync_copy(pe_obs_hbm.at[pl.ds(0, _LO)], pe_obs_v)
    pltpu.sync_copy(pe_act_hbm, pe_act_v)
    pltpu.sync_copy(pe_time_hbm, pe_time_v)
    pltpu.sync_copy(w_sp_hbm.at[pl.ds(0, 1)], sp_v)
    pltpu.sync_copy(pe_sp_hbm, pesp_v)
    for c in range(_D // 16):
        sp_v[0, pl.ds(c * 16, 16)] = (
            sp_v[0, pl.ds(c * 16, 16)] + pesp_v[0, pl.ds(c * 16, 16)]
        )

    def fire(g, slot):
        pltpu.make_async_copy(
            w_obs_hbm.at[idx_obs_v.at[pl.ds(g * _OBS_C, _OBS_C)]],
            obs_rows.at[slot], sem_go[slot]).start()
        pltpu.make_async_copy(
            w_act_hbm.at[idx_act_v.at[pl.ds(g * _ACT_C, _ACT_C)]],
            act_rows.at[slot], sem_ga[slot]).start()

    def compute_store(g, slot):
        pltpu.make_async_copy(
            w_obs_hbm.at[idx_obs_v.at[pl.ds(g * _OBS_C, _OBS_C)]],
            obs_rows.at[slot], sem_go[slot]).wait()
        pltpu.make_async_copy(
            w_act_hbm.at[idx_act_v.at[pl.ds(g * _ACT_C, _ACT_C)]],
            act_rows.at[slot], sem_ga[slot]).wait()

        @pl.when(g >= 2)
        def _():
            pltpu.make_async_copy(
                out_buf.at[slot],
                out_hbm.at[pl.ds(out_base, _OUT_C)], sem_out[slot]).wait()

        t0 = lax.rem(g * _G, _T)
        for j in range(_G):
            t = lax.rem(t0 + j, _T)
            tv = [pe_time_v[t, pl.ds(c * 16, 16)] for c in range(_D // 16)]
            for s in range(_LO):
                for c in range(_D // 16):
                    out_buf[slot, j * _LT + s, pl.ds(c * 16, 16)] = (
                        obs_rows[slot, j * _LO + s, pl.ds(c * 16, 16)]
                        + pe_obs_v[s, pl.ds(c * 16, 16)] + tv[c]
                    )
            for c in range(_D // 16):
                out_buf[slot, j * _LT + _LO, pl.ds(c * 16, 16)] = (
                    sp_v[0, pl.ds(c * 16, 16)] + tv[c]
                )
            for a in range(_LA):
                for c in range(_D // 16):
                    out_buf[slot, j * _LT + _LO + 1 + a, pl.ds(c * 16, 16)] = (
                        act_rows[slot, j * _LA + a, pl.ds(c * 16, 16)]
                        + pe_act_v[a, pl.ds(c * 16, 16)] + tv[c]
                    )

        pltpu.make_async_copy(
            out_buf.at[slot],
            out_hbm.at[pl.ds(out_base + g * _OUT_C, _OUT_C)],
            sem_out[slot]).start()

    fire(0, 0)

    def outer(i, carry):
        g0 = 2 * i
        fire(g0 + 1, 1)
        compute_store(g0, 0)

        @pl.when(g0 + 2 < _NCHUNK)
        def _():
            fire(g0 + 2, 0)

        compute_store(g0 + 1, 1)
        return carry

    lax.fori_loop(0, _NCHUNK // 2, outer, 0)

    for slot in range(2):
        pltpu.make_async_copy(
            out_buf.at[slot],
            out_hbm.at[pl.ds(out_base, _OUT_C)], sem_out[slot]).wait()


def kernel(obs_tokens, act_tokens, W_obs, W_act, W_special,
           PE_obs, PE_act, PE_special, PE_time):
    obs_idx = obs_tokens.reshape(-1)
    act_idx = act_tokens.reshape(-1)
    out = _episode_sc(obs_idx, act_idx, W_obs, W_act, W_special,
                      PE_obs, PE_act, PE_special, PE_time)
    return out.reshape(_B, _T * _LT, _D)
